# baseline (speedup 1.0000x reference)
import jax
import jax.numpy as jnp
from jax.experimental import pallas as pl
from jax.experimental.pallas import tpu as pltpu

_P = 8
_CH = 4


def _med3(a, b, c):
    return jnp.maximum(jnp.minimum(a, b), jnp.minimum(jnp.maximum(a, b), c))


def _median3x3_kernel(x_ref, o_ref):
    P, H, W = x_ref.shape
    Wc = W // _CH
    dt = o_ref.dtype

    for p in range(P):
        x = x_ref[p]
        zcol = jnp.zeros((H, 1), dt)
        xl = jnp.concatenate([zcol, x[:, :-1]], axis=1)
        xr = jnp.concatenate([x[:, 1:], zcol], axis=1)

        mnh = jnp.minimum(x, xr)
        mxh = jnp.maximum(x, xr)
        lo = jnp.minimum(xl, mnh)
        hi = jnp.maximum(xl, mxh)
        mid = jnp.maximum(jnp.minimum(xl, mxh), mnh)

        zrow = jnp.zeros((1, Wc), dt)

        def shu(a):
            return jnp.concatenate([zrow, a[:-1, :]], axis=0)

        def shd(a):
            return jnp.concatenate([a[1:, :], zrow], axis=0)

        for c in range(_CH):
            cs = slice(Wc * c, Wc * (c + 1))
            loc, midc, hic = lo[:, cs], mid[:, cs], hi[:, cs]
            mx = jnp.maximum(jnp.maximum(shu(loc), loc), shd(loc))
            mn = jnp.minimum(jnp.minimum(shu(hic), hic), shd(hic))
            md = _med3(shu(midc), midc, shd(midc))
            o_ref[p, :, cs] = _med3(mx, md, mn)


@jax.jit
def kernel(x):
    B, C, H, W = x.shape
    N = B * C
    xf = x.reshape(N, H, W)
    out = pl.pallas_call(
        _median3x3_kernel,
        grid=(N // _P,),
        in_specs=[pl.BlockSpec((_P, H, W), lambda i: (i, 0, 0))],
        out_specs=pl.BlockSpec((_P, H, W), lambda i: (i, 0, 0)),
        out_shape=jax.ShapeDtypeStruct((N, H, W), x.dtype),
        compiler_params=pltpu.CompilerParams(
            dimension_semantics=("parallel",),
        ),
    )(xf)
    return out.reshape(B, C, H, W)


# speedup vs baseline: 1.0101x; 1.0101x over previous
import jax
import jax.numpy as jnp
from jax.experimental import pallas as pl
from jax.experimental.pallas import tpu as pltpu

_P = 4
_CH = 4


def _med3(a, b, c):
    return jnp.maximum(jnp.minimum(a, b), jnp.minimum(jnp.maximum(a, b), c))


def _median3x3_kernel(x_ref, o_ref):
    P, H, W = x_ref.shape
    Wc = W // _CH
    dt = o_ref.dtype

    for p in range(P):
        x = x_ref[p]
        zcol = jnp.zeros((H, 1), dt)
        xl = jnp.concatenate([zcol, x[:, :-1]], axis=1)
        xr = jnp.concatenate([x[:, 1:], zcol], axis=1)

        mnh = jnp.minimum(x, xr)
        mxh = jnp.maximum(x, xr)
        lo = jnp.minimum(xl, mnh)
        hi = jnp.maximum(xl, mxh)
        mid = jnp.maximum(jnp.minimum(xl, mxh), mnh)

        zrow = jnp.zeros((1, Wc), dt)

        def shu(a):
            return jnp.concatenate([zrow, a[:-1, :]], axis=0)

        def shd(a):
            return jnp.concatenate([a[1:, :], zrow], axis=0)

        for c in range(_CH):
            cs = slice(Wc * c, Wc * (c + 1))
            loc, midc, hic = lo[:, cs], mid[:, cs], hi[:, cs]
            mx = jnp.maximum(jnp.maximum(shu(loc), loc), shd(loc))
            mn = jnp.minimum(jnp.minimum(shu(hic), hic), shd(hic))
            md = _med3(shu(midc), midc, shd(midc))
            o_ref[p, :, cs] = _med3(mx, md, mn)


@jax.jit
def kernel(x):
    B, C, H, W = x.shape
    N = B * C
    xf = x.reshape(N, H, W)
    out = pl.pallas_call(
        _median3x3_kernel,
        grid=(N // _P,),
        in_specs=[pl.BlockSpec((_P, H, W), lambda i: (i, 0, 0))],
        out_specs=pl.BlockSpec((_P, H, W), lambda i: (i, 0, 0)),
        out_shape=jax.ShapeDtypeStruct((N, H, W), x.dtype),
        compiler_params=pltpu.CompilerParams(
            dimension_semantics=("parallel",),
        ),
    )(xf)
    return out.reshape(B, C, H, W)
